# baseline (speedup 1.0000x reference)
import functools

import jax
import jax.numpy as jnp
from jax import lax
from jax.experimental import pallas as pl
from jax.experimental.pallas import tpu as pltpu
from jax.experimental.pallas import tpu_sc as plsc

V = 1000000
D = 64
F = 32
B = 1024
LU = 50
K = 20
LN = 20

NC = 2
NS = 16
NW = NC * NS
L = 16

CH = 80

U_ROWS = B * LU
N_ROWS = B * K * LN
U_CHUNKS_W = U_ROWS // (CH * NW)
N_CHUNKS_W = N_ROWS // (CH * NW)
U_BAGS_W = B // NW
N_BAGS_W = (B * K) // NW

_mesh = plsc.VectorSubcoreMesh(
    core_axis_name="c", subcore_axis_name="s", num_cores=NC, num_subcores=NS
)


_GATHER_DN = lax.GatherDimensionNumbers(
    offset_dims=(), collapsed_slice_dims=(0,), start_index_map=(0,)
)


def _splat(wvec, t):
    return lax.gather(
        wvec,
        jnp.full((L, 1), t, jnp.int32),
        _GATHER_DN,
        (1,),
        mode=lax.GatherScatterMode.PROMISE_IN_BOUNDS,
    )


@functools.partial(
    pl.kernel,
    out_type=(
        jax.ShapeDtypeStruct((B, D), jnp.float32),
        jax.ShapeDtypeStruct((B * K, D), jnp.float32),
    ),
    mesh=_mesh,
    scratch_types=[
        pltpu.VMEM((U_CHUNKS_W, CH), jnp.int32),
        pltpu.VMEM((U_CHUNKS_W, CH), jnp.float32),
        pltpu.VMEM((N_CHUNKS_W, CH), jnp.int32),
        pltpu.VMEM((N_CHUNKS_W, CH), jnp.float32),
        pltpu.VMEM((CH, D), jnp.float32),
        pltpu.VMEM((U_BAGS_W, D), jnp.float32),
        pltpu.VMEM((N_BAGS_W, D), jnp.float32),
        pltpu.SemaphoreType.DMA,
    ],
    compiler_params=pltpu.CompilerParams(use_tc_tiling_on_sc=False),
)
def _sc_bag_sums(ui, uw, ni, nw, table, out_u, out_n,
                 ui_v, uw_v, ni_v, nw_v, rbuf, ou_v, on_v, sem):
    wid = lax.axis_index("s") * NC + lax.axis_index("c")

    pltpu.sync_copy(ui.at[wid], ui_v)
    pltpu.sync_copy(uw.at[wid], uw_v)
    pltpu.sync_copy(ni.at[wid], ni_v)
    pltpu.sync_copy(nw.at[wid], nw_v)

    zero = jnp.zeros((L,), jnp.float32)

    def zero_out(i, _):
        for cc in range(D // L):
            ou_v[lax.rem(i, U_BAGS_W), pl.ds(cc * L, L)] = zero
            on_v[i, pl.ds(cc * L, L)] = zero
        return 0

    lax.fori_loop(0, N_BAGS_W, zero_out, 0)

    def phase(idx_v, w_v, chunks_w, bag_len, out_v):
        rows_w = chunks_w * CH

        def chunk_body(c, _):
            cp = pltpu.async_copy(table.at[idx_v.at[c]], rbuf, sem)
            cp.wait()

            def group_body(g, _):
                wvec = w_v[c, pl.ds(g * L, L)]
                row0 = wid * rows_w + c * CH + g * L
                for t in range(L):
                    w = _splat(wvec, t)
                    bag = lax.div(row0 + t, bag_len) - wid * (rows_w // bag_len)
                    r = g * L + t
                    for cc in range(D // L):
                        plsc.addupdate(
                            out_v.at[bag, pl.ds(cc * L, L)],
                            w * rbuf[r, pl.ds(cc * L, L)],
                        )
                return 0

            lax.fori_loop(0, CH // L, group_body, 0)
            return 0

        lax.fori_loop(0, chunks_w, chunk_body, 0)

    phase(ui_v, uw_v, U_CHUNKS_W, LU, ou_v)
    phase(ni_v, nw_v, N_CHUNKS_W, LN, on_v)

    pltpu.sync_copy(ou_v, out_u.at[pl.ds(wid * U_BAGS_W, U_BAGS_W)])
    pltpu.sync_copy(on_v, out_n.at[pl.ds(wid * N_BAGS_W, N_BAGS_W)])


def _mlp_body(ue_ref, ne_ref, w3t_ref, b3_ref, w4t_ref, b4_ref, out_ref):
    w3t = w3t_ref[...]
    b3 = b3_ref[...]
    w4t = w4t_ref[...]
    b4 = b4_ref[...]
    uy = jnp.tanh(
        jnp.tanh(jnp.dot(ue_ref[...], w3t, preferred_element_type=jnp.float32) + b3)
        @ w4t
        + b4
    )
    ny = jnp.tanh(
        jnp.tanh(jnp.dot(ne_ref[...], w3t, preferred_element_type=jnp.float32) + b3)
        @ w4t
        + b4
    )
    un = uy * lax.rsqrt(jnp.sum(uy * uy, axis=1, keepdims=True))
    nn = ny * lax.rsqrt(jnp.sum(ny * ny, axis=1, keepdims=True))
    nn3 = nn.reshape(B, K, F)
    out_ref[...] = jnp.sum(un[:, None, :] * nn3, axis=2)


def _mlp(ue, ne, w3t, b3, w4t, b4):
    return pl.pallas_call(
        _mlp_body,
        out_shape=jax.ShapeDtypeStruct((B, K), jnp.float32),
    )(ue, ne, w3t, b3, w4t, b4)


def kernel(user_indices, user_weights, user_seq_len, news_indices, news_weights,
           news_seq_len, emb_table, W3, b3, W4, b4):
    del user_seq_len, news_seq_len
    ui = user_indices.astype(jnp.int32).reshape(NW, U_CHUNKS_W, CH)
    uw = user_weights.reshape(NW, U_CHUNKS_W, CH)
    ni = news_indices.astype(jnp.int32).reshape(NW, N_CHUNKS_W, CH)
    nw = news_weights.reshape(NW, N_CHUNKS_W, CH)
    ue, ne = _sc_bag_sums(ui, uw, ni, nw, emb_table)
    return _mlp(ue, ne, W3.T, b3.reshape(1, D), W4.T, b4.reshape(1, F))


# speedup vs baseline: 1.2996x; 1.2996x over previous
import functools

import jax
import jax.numpy as jnp
from jax import lax
from jax.experimental import pallas as pl
from jax.experimental.pallas import tpu as pltpu
from jax.experimental.pallas import tpu_sc as plsc

V = 1000000
D = 64
F = 32
B = 1024
LU = 50
K = 20
LN = 20

NC = 2
NS = 16
NW = NC * NS
L = 16

CH = 80

UB_W = B // NW
NB_W = (B * K) // NW
U_ROWS_W = UB_W * LU
N_ROWS_W = NB_W * LN
U_CHUNKS = U_ROWS_W // CH
N_CHUNKS = N_ROWS_W // CH
NBAGS_CH = CH // LN

_mesh = plsc.VectorSubcoreMesh(
    core_axis_name="c", subcore_axis_name="s", num_cores=NC, num_subcores=NS
)

_GATHER_DN = lax.GatherDimensionNumbers(
    offset_dims=(), collapsed_slice_dims=(0,), start_index_map=(0,)
)


def _splat(wvec, t):
    return lax.gather(
        wvec,
        jnp.full((L, 1), t, jnp.int32),
        _GATHER_DN,
        (1,),
        mode=lax.GatherScatterMode.PROMISE_IN_BOUNDS,
    )


def _full(x):
    return jnp.full((L,), x, jnp.int32)


@functools.partial(
    pl.kernel,
    out_type=(
        jax.ShapeDtypeStruct((B, D), jnp.float32),
        jax.ShapeDtypeStruct((B * K, D), jnp.float32),
    ),
    mesh=_mesh,
    scratch_types=[
        pltpu.VMEM((UB_W, LU), jnp.int32),
        pltpu.VMEM((UB_W, LU), jnp.float32),
        pltpu.VMEM((UB_W, K, LN), jnp.int32),
        pltpu.VMEM((UB_W, K, LN), jnp.float32),
        pltpu.VMEM((U_ROWS_W,), jnp.int32),
        pltpu.VMEM((N_ROWS_W,), jnp.int32),
        pltpu.VMEM((CH, D), jnp.float32),
        pltpu.VMEM((CH, D), jnp.float32),
        pltpu.VMEM((UB_W, D), jnp.float32),
        pltpu.VMEM((NB_W, D), jnp.float32),
        pltpu.SemaphoreType.DMA,
        pltpu.SemaphoreType.DMA,
    ],
    compiler_params=pltpu.CompilerParams(
        use_tc_tiling_on_sc=False, needs_layout_passes=False
    ),
)
def _sc_bag_sums(ui, uw, ni, nw, table, out_u, out_n,
                 ui_v, uw_v, ni_v, nw_v, uif, nif, rb0, rb1, ou_v, on_v,
                 sem0, sem1):
    wid = lax.axis_index("s") * NC + lax.axis_index("c")
    iota = lax.iota(jnp.int32, L)
    zero = jnp.zeros((L,), jnp.float32)

    pltpu.sync_copy(ui.at[pl.ds(wid * UB_W, UB_W)], ui_v)
    pltpu.sync_copy(uw.at[pl.ds(wid * UB_W, UB_W)], uw_v)
    pltpu.sync_copy(ni.at[pl.ds(wid * UB_W, UB_W)], ni_v)
    pltpu.sync_copy(nw.at[pl.ds(wid * UB_W, UB_W)], nw_v)

    def rep_u(i, _):
        f = i * L + iota
        b = lax.div(f, _full(LU))
        t = f - b * LU
        uif[pl.ds(pl.multiple_of(i * L, L), L)] = plsc.load_gather(ui_v, [b, t])
        return 0

    lax.fori_loop(0, U_ROWS_W // L, rep_u, 0)

    def rep_n(i, _):
        f = i * L + iota
        b = lax.div(f, _full(K * LN))
        r = f - b * (K * LN)
        k = lax.div(r, _full(LN))
        t = r - k * LN
        nif[pl.ds(pl.multiple_of(i * L, L), L)] = plsc.load_gather(ni_v, [b, k, t])
        return 0

    lax.fori_loop(0, N_ROWS_W // L, rep_n, 0)

    def zbody(i, _):
        for cc in range(D // L):
            ou_v[i, pl.ds(cc * L, L)] = zero
        return 0

    lax.fori_loop(0, UB_W, zbody, 0)

    def gstart(idxf, c, rb, sem):
        pltpu.async_copy(
            table.at[idxf.at[pl.ds(pl.multiple_of(c * CH, CH), CH)]], rb, sem
        )

    def gwait(idxf, c, rb, sem):
        pltpu.make_async_copy(
            table.at[idxf.at[pl.ds(pl.multiple_of(c * CH, CH), CH)]], rb, sem
        ).wait()

    def ucompute(c, rb):
        base = c * CH
        for g in range(CH // L):
            f = base + g * L + iota
            bv = lax.div(f, _full(LU))
            tv = f - bv * LU
            wvec = plsc.load_gather(uw_v, [bv, tv])
            for tt in range(L):
                w = _splat(wvec, tt)
                bag = lax.div(base + g * L + tt, LU)
                r = g * L + tt
                for cc in range(D // L):
                    plsc.addupdate(
                        ou_v.at[bag, pl.ds(cc * L, L)],
                        w * rb[r, pl.ds(cc * L, L)],
                    )

    def ncompute(c, rb):
        for jj in range(NBAGS_CH):
            j = c * NBAGS_CH + jj
            b = lax.div(j, K)
            k = j - b * K
            w0 = plsc.load_gather(nw_v, [_full(b), _full(k), iota])
            w1 = plsc.load_gather(nw_v, [_full(b), _full(k), iota + (LN - L)])
            acc = [zero] * (D // L)
            for t in range(LN):
                w = _splat(w0, t) if t < L else _splat(w1, t - (LN - L))
                r = jj * LN + t
                for cc in range(D // L):
                    acc[cc] = acc[cc] + w * rb[r, pl.ds(cc * L, L)]
            for cc in range(D // L):
                on_v[j, pl.ds(cc * L, L)] = acc[cc]

    def run_phase(idxf, nch, compute):
        gstart(idxf, 0, rb0, sem0)

        def body(c2, _):
            c = c2 * 2
            gstart(idxf, c + 1, rb1, sem1)
            gwait(idxf, c, rb0, sem0)
            compute(c, rb0)

            @pl.when(c + 2 < nch)
            def _():
                gstart(idxf, c + 2, rb0, sem0)

            gwait(idxf, c + 1, rb1, sem1)
            compute(c + 1, rb1)
            return 0

        lax.fori_loop(0, nch // 2, body, 0)

    run_phase(uif, U_CHUNKS, ucompute)
    run_phase(nif, N_CHUNKS, ncompute)

    pltpu.sync_copy(ou_v, out_u.at[pl.ds(wid * UB_W, UB_W)])
    pltpu.sync_copy(on_v, out_n.at[pl.ds(wid * NB_W, NB_W)])


def _mlp_body(ue_ref, ne_ref, w3t_ref, b3_ref, w4t_ref, b4_ref, out_ref):
    w3t = w3t_ref[...]
    b3 = b3_ref[...]
    w4t = w4t_ref[...]
    b4 = b4_ref[...]
    uy = jnp.tanh(
        jnp.tanh(jnp.dot(ue_ref[...], w3t, preferred_element_type=jnp.float32) + b3)
        @ w4t
        + b4
    )
    ny = jnp.tanh(
        jnp.tanh(jnp.dot(ne_ref[...], w3t, preferred_element_type=jnp.float32) + b3)
        @ w4t
        + b4
    )
    un = uy * lax.rsqrt(jnp.sum(uy * uy, axis=1, keepdims=True))
    nn = ny * lax.rsqrt(jnp.sum(ny * ny, axis=1, keepdims=True))
    nn3 = nn.reshape(B, K, F)
    out_ref[...] = jnp.sum(un[:, None, :] * nn3, axis=2)


def _mlp(ue, ne, w3t, b3, w4t, b4):
    return pl.pallas_call(
        _mlp_body,
        out_shape=jax.ShapeDtypeStruct((B, K), jnp.float32),
    )(ue, ne, w3t, b3, w4t, b4)


def kernel(user_indices, user_weights, user_seq_len, news_indices, news_weights,
           news_seq_len, emb_table, W3, b3, W4, b4):
    del user_seq_len, news_seq_len
    ue, ne = _sc_bag_sums(
        user_indices.astype(jnp.int32),
        user_weights,
        news_indices.astype(jnp.int32),
        news_weights,
        emb_table,
    )
    return _mlp(ue, ne, W3.T, b3.reshape(1, D), W4.T, b4.reshape(1, F))
